# baseline (speedup 1.0000x reference)
import functools

import jax
import jax.numpy as jnp
from jax import lax
from jax.experimental import pallas as pl
from jax.experimental.pallas import tpu as pltpu
from jax.experimental.pallas import tpu_sc as plsc

N_ATOMS = 1024
R_MAX2 = 100.0


_PB = N_ATOMS
_PI = 256
_PS = 512
_PBB = 256
_OFF_B0 = 0
_OFF_B1 = _OFF_B0 + _PB
_OFF_BT = _OFF_B1 + _PB
_OFF_A0 = _OFF_BT + _PB
_OFF_A1 = _OFF_A0 + _PB
_OFF_A2 = _OFF_A1 + _PB
_OFF_AT = _OFF_A2 + _PB
_OFF_I0 = _OFF_AT + _PB
_OFF_I1 = _OFF_I0 + _PI
_OFF_I2 = _OFF_I1 + _PI
_OFF_I3 = _OFF_I2 + _PI
_OFF_IT = _OFF_I3 + _PI
_OFF_S0 = _OFF_IT + _PI
_OFF_S1 = _OFF_S0 + _PS
_OFF_S2 = _OFF_S1 + _PS
_OFF_S3 = _OFF_S2 + _PS
_OFF_G0 = _OFF_S3 + _PS
_OFF_G1 = _OFF_G0 + _PBB
_OFF_G2 = _OFF_G1 + _PBB
_OFF_G3 = _OFF_G2 + _PBB
_IDX_LEN = _OFF_G3 + _PBB

_OFF_TBK = 0
_OFF_TBE = 16
_OFF_TAK = 32
_OFF_TAE = 48
_OFF_TDK = 64
_OFF_TDE = 80
_OFF_SC = 96
_OFF_SP = _OFF_SC + _PS
_OFF_SM = _OFF_SP + _PS
_OFF_GC = _OFF_SM + _PS
_OFF_GP = _OFF_GC + _PBB
_OFF_GM = _OFF_GP + _PBB
_FLT_LEN = _OFF_GM + _PBB

_PI_F = 3.14159265358979323846


def _rsqrt16(s):
    s = jnp.maximum(s, 1e-30)
    i = lax.bitcast_convert_type(s, jnp.int32)
    i = 0x5F3759DF - lax.shift_right_logical(i, 1)
    y = lax.bitcast_convert_type(i, jnp.float32)
    for _ in range(3):
        y = y * (1.5 - 0.5 * s * y * y)
    return y


def _sqrt16(s):
    return s * _rsqrt16(s)


def _atan16(a):
    s = a * a
    p = jnp.float32(0.0028662257)
    p = p * s + jnp.float32(-0.0161657367)
    p = p * s + jnp.float32(0.0429096138)
    p = p * s + jnp.float32(-0.0752896400)
    p = p * s + jnp.float32(0.1065626393)
    p = p * s + jnp.float32(-0.1420889944)
    p = p * s + jnp.float32(0.1999355085)
    p = p * s + jnp.float32(-0.3333314528)
    p = p * s + jnp.float32(1.0)
    return a * p


def _atan2_16(y, x):
    ax = jnp.abs(x)
    ay = jnp.abs(y)
    mx = jnp.maximum(ax, ay)
    mn = jnp.minimum(ax, ay)
    a = mn / jnp.maximum(mx, 1e-30)
    r = _atan16(a)
    r = jnp.where(ay > ax, _PI_F / 2.0 - r, r)
    r = jnp.where(x < 0.0, _PI_F - r, r)
    return jnp.where(y < 0.0, -r, r)


def _cos16(t):
    q = t * jnp.float32(1.0 / (2.0 * _PI_F))
    q = q + jnp.where(q >= 0.0, 0.5, -0.5)
    k = q.astype(jnp.int32).astype(jnp.float32)
    t = t - k * jnp.float32(2.0 * _PI_F)
    s = t * t
    p = jnp.float32(1.0 / 479001600.0)
    p = p * s + jnp.float32(-1.0 / 3628800.0)
    p = p * s + jnp.float32(1.0 / 40320.0)
    p = p * s + jnp.float32(-1.0 / 720.0)
    p = p * s + jnp.float32(1.0 / 24.0)
    p = p * s + jnp.float32(-0.5)
    p = p * s + jnp.float32(1.0)
    return p


def _gather3(pos_v, idx):
    base = idx * 3
    return (plsc.load_gather(pos_v, [base]),
            plsc.load_gather(pos_v, [base + 1]),
            plsc.load_gather(pos_v, [base + 2]))


def _dih16(p0, p1, p2, p3):
    b0 = tuple(p0[c] - p1[c] for c in range(3))
    b1 = tuple(p2[c] - p1[c] for c in range(3))
    b2 = tuple(p3[c] - p2[c] for c in range(3))
    s1 = b1[0] * b1[0] + b1[1] * b1[1] + b1[2] * b1[2]
    n1 = _sqrt16(s1)
    m = 1.0 / (n1 + 1e-9)
    b1n = tuple(b1[c] * m for c in range(3))
    d0 = b0[0] * b1n[0] + b0[1] * b1n[1] + b0[2] * b1n[2]
    d2 = b2[0] * b1n[0] + b2[1] * b1n[1] + b2[2] * b1n[2]
    v = tuple(b0[c] - d0 * b1n[c] for c in range(3))
    w = tuple(b2[c] - d2 * b1n[c] for c in range(3))
    xx = v[0] * w[0] + v[1] * w[1] + v[2] * w[2]
    c0 = b1n[1] * v[2] - b1n[2] * v[1]
    c1 = b1n[2] * v[0] - b1n[0] * v[2]
    c2 = b1n[0] * v[1] - b1n[1] * v[0]
    yy = c0 * w[0] + c1 * w[1] + c2 * w[2]
    return _atan2_16(yy, xx)


def _sc_body(pos_hbm, idx_hbm, flt_hbm, out_hbm, pos_v, idx_v, flt_v, acc_v):
    nc = 2
    wid = lax.axis_index("s") * nc + lax.axis_index("c")
    frame = wid // 2
    half = wid % 2
    pltpu.sync_copy(pos_hbm.at[frame], pos_v)
    pltpu.sync_copy(idx_hbm, idx_v)
    pltpu.sync_copy(flt_hbm, flt_v)
    iota = lax.iota(jnp.int32, 16)

    def bonds(i, acc):
        t = half * (_PB // 2) + i * 16 + iota
        ia = plsc.load_gather(idx_v, [t + _OFF_B0])
        ib = plsc.load_gather(idx_v, [t + _OFF_B1])
        ty = plsc.load_gather(idx_v, [t + _OFF_BT])
        k = plsc.load_gather(flt_v, [ty + _OFF_TBK])
        e = plsc.load_gather(flt_v, [ty + _OFF_TBE])
        pa = _gather3(pos_v, ia)
        pb = _gather3(pos_v, ib)
        dx, dy, dz = (pa[c] - pb[c] for c in range(3))
        d = _sqrt16(dx * dx + dy * dy + dz * dz)
        return acc + 0.5 * k * (d - e) * (d - e)

    def angles(i, acc):
        t = half * (_PB // 2) + i * 16 + iota
        i0 = plsc.load_gather(idx_v, [t + _OFF_A0])
        i1 = plsc.load_gather(idx_v, [t + _OFF_A1])
        i2 = plsc.load_gather(idx_v, [t + _OFF_A2])
        ty = plsc.load_gather(idx_v, [t + _OFF_AT])
        k = plsc.load_gather(flt_v, [ty + _OFF_TAK])
        e = plsc.load_gather(flt_v, [ty + _OFF_TAE])
        p0 = _gather3(pos_v, i0)
        p1 = _gather3(pos_v, i1)
        p2 = _gather3(pos_v, i2)
        v1 = tuple(p0[c] - p1[c] for c in range(3))
        v2 = tuple(p2[c] - p1[c] for c in range(3))
        s1 = v1[0] * v1[0] + v1[1] * v1[1] + v1[2] * v1[2]
        s2 = v2[0] * v2[0] + v2[1] * v2[1] + v2[2] * v2[2]
        n1 = _sqrt16(s1) + 1e-9
        n2 = _sqrt16(s2) + 1e-9
        dot = v1[0] * v2[0] + v1[1] * v2[1] + v1[2] * v2[2]
        c = jnp.clip(dot / (n1 * n2), -1.0 + 1e-7, 1.0 - 1e-7)
        ang = _atan2_16(_sqrt16((1.0 - c) * (1.0 + c)), c)
        return acc + 0.5 * k * (ang - e) * (ang - e)

    def improper(i, acc):
        t = half * (_PI // 2) + i * 16 + iota
        p0 = _gather3(pos_v, plsc.load_gather(idx_v, [t + _OFF_I0]))
        p1 = _gather3(pos_v, plsc.load_gather(idx_v, [t + _OFF_I1]))
        p2 = _gather3(pos_v, plsc.load_gather(idx_v, [t + _OFF_I2]))
        p3 = _gather3(pos_v, plsc.load_gather(idx_v, [t + _OFF_I3]))
        ty = plsc.load_gather(idx_v, [t + _OFF_IT])
        k = plsc.load_gather(flt_v, [ty + _OFF_TDK])
        e = plsc.load_gather(flt_v, [ty + _OFF_TDE])
        phi = _dih16(p0, p1, p2, p3)
        return acc + 0.5 * k * (phi - e) * (phi - e)

    def sbbs(i, acc):
        t = half * (_PS // 2) + i * 16 + iota
        p0 = _gather3(pos_v, plsc.load_gather(idx_v, [t + _OFF_S0]))
        p1 = _gather3(pos_v, plsc.load_gather(idx_v, [t + _OFF_S1]))
        p2 = _gather3(pos_v, plsc.load_gather(idx_v, [t + _OFF_S2]))
        p3 = _gather3(pos_v, plsc.load_gather(idx_v, [t + _OFF_S3]))
        const = plsc.load_gather(flt_v, [t + _OFF_SC])
        phase = plsc.load_gather(flt_v, [t + _OFF_SP])
        mul = plsc.load_gather(flt_v, [t + _OFF_SM])
        phi = _dih16(p0, p1, p2, p3)
        return acc + const * (1.0 + _cos16(mul * phi - phase))

    def bbbb(i, acc):
        t = half * (_PBB // 2) + i * 16 + iota
        p0 = _gather3(pos_v, plsc.load_gather(idx_v, [t + _OFF_G0]))
        p1 = _gather3(pos_v, plsc.load_gather(idx_v, [t + _OFF_G1]))
        p2 = _gather3(pos_v, plsc.load_gather(idx_v, [t + _OFF_G2]))
        p3 = _gather3(pos_v, plsc.load_gather(idx_v, [t + _OFF_G3]))
        const = plsc.load_gather(flt_v, [t + _OFF_GC])
        phase = plsc.load_gather(flt_v, [t + _OFF_GP])
        mul = plsc.load_gather(flt_v, [t + _OFF_GM])
        phi = _dih16(p0, p1, p2, p3)
        return acc + const * (1.0 + _cos16(mul * phi - phase))

    acc = jnp.zeros((16,), jnp.float32)
    acc = lax.fori_loop(0, _PB // 32, bonds, acc)
    acc = lax.fori_loop(0, _PB // 32, angles, acc)
    acc = lax.fori_loop(0, _PI // 32, improper, acc)
    acc = lax.fori_loop(0, _PS // 32, sbbs, acc)
    acc = lax.fori_loop(0, _PBB // 32, bbbb, acc)
    acc_v[...] = acc
    pltpu.sync_copy(acc_v, out_hbm.at[wid])


def _sc_bonded(pos_flat, idx_buf, flt_buf):
    mesh = plsc.VectorSubcoreMesh(core_axis_name="c", subcore_axis_name="s")
    return pl.kernel(
        _sc_body,
        out_type=jax.ShapeDtypeStruct((32, 16), jnp.float32),
        mesh=mesh,
        scratch_types=[
            pltpu.VMEM((3 * N_ATOMS,), jnp.float32),
            pltpu.VMEM((_IDX_LEN,), jnp.int32),
            pltpu.VMEM((_FLT_LEN,), jnp.float32),
            pltpu.VMEM((16,), jnp.float32),
        ],
        compiler_params=pltpu.CompilerParams(needs_layout_passes=False),
    )(pos_flat, idx_buf, flt_buf)


def _onehot_lookup(type_plane, table_ref, n):
    out = jnp.zeros_like(type_plane, dtype=jnp.float32)
    for k in range(n):
        out = jnp.where(type_plane == k, table_ref[0, k], out)
    return out


def _pw_kernel(pos_ref, bt_ref, br_ref, disp_ref, out_ref):
    x = pos_ref[0]
    n = x.shape[1]
    bead_types = bt_ref[0:1, :]
    radii_row = _onehot_lookup(bead_types, br_ref, br_ref.shape[1])
    sq_row = jnp.sum(x * x, axis=0, keepdims=True)
    ones = jnp.ones((1, n), jnp.float32)
    lhs_r2 = jnp.concatenate([x, sq_row, ones], axis=0)
    rhs_r2 = jnp.concatenate([-2.0 * x, ones, sq_row], axis=0)
    lhs_sg = jnp.concatenate([radii_row, ones], axis=0)
    rhs_sg = jnp.concatenate([ones, radii_row], axis=0)
    tile = 128
    row = lax.broadcasted_iota(jnp.int32, (tile, tile), 0)
    col = lax.broadcasted_iota(jnp.int32, (tile, tile), 1)
    tri = col > row + 2
    tri2 = col + tile > row + 2
    e_nb = jnp.float32(0.0)
    for ti in range(n // tile):
        lo = ti * tile

        def rep_block(cl, cw, extra_mask=None):
            r2 = lax.dot_general(lhs_r2[:, lo:lo + tile],
                                 rhs_r2[:, cl:cl + cw],
                                 (((0,), (0,)), ((), ())),
                                 preferred_element_type=jnp.float32)
            sig = lax.dot_general(lhs_sg[:, lo:lo + tile],
                                  rhs_sg[:, cl:cl + cw],
                                  (((0,), (0,)), ((), ())),
                                  preferred_element_type=jnp.float32)
            r2 = jnp.maximum(r2, 1e-6)
            sig2 = sig * sig
            t3 = sig2 / (r2 + sig2)
            rep = t3 * t3 * t3
            pmask = r2 < R_MAX2
            if extra_mask is not None:
                pmask = pmask & extra_mask
            return jnp.sum(jnp.where(pmask, rep, 0.0))

        e_nb = e_nb + rep_block(lo, tile, tri)
        if lo + tile < n:
            e_nb = e_nb + rep_block(lo + tile, tile, tri2)
        if lo + 2 * tile < n:
            e_nb = e_nb + rep_block(lo + 2 * tile, n - lo - 2 * tile)
    out_ref[...] = jnp.full((1, 1, 1), disp_ref[0, 0] * e_nb, jnp.float32)


def _pairwise(pos_t, bead_types, bead_radii, disp):
    b, _, n = pos_t.shape
    out = pl.pallas_call(
        _pw_kernel,
        grid=(b,),
        in_specs=[
            pl.BlockSpec((1, 3, n), lambda i: (i, 0, 0)),
            pl.BlockSpec((1, n), lambda i: (0, 0)),
            pl.BlockSpec((1, 16), lambda i: (0, 0)),
            pl.BlockSpec((1, 1), lambda i: (0, 0)),
        ],
        out_specs=pl.BlockSpec((1, 1, 1), lambda i: (i, 0, 0)),
        out_shape=jax.ShapeDtypeStruct((b, 1, 1), jnp.float32),
    )(pos_t, bead_types.reshape(1, n), bead_radii.reshape(1, -1),
      disp.reshape(1, 1))
    return out.reshape(b)


def _pad_to(a, n, value=0):
    return jnp.pad(a, (0, n - a.shape[0]), constant_values=value)


@jax.jit
def kernel(pos, bond_k, angle_k, dih_k, sbbs_phase, sbbs_const, bbbb_phase,
           bbbb_const, bead_radii, dispertion_const, bond_equ, angle_equ,
           dih_equ, bond_indices, bond_type, angle_indices, angle_type,
           improper_indices, dih_type, sbbs_indices, sbbs_mul, bbbb_indices,
           bbbb_mul, bead_types):
    b, n, _ = pos.shape
    pos_t = jnp.transpose(pos, (0, 2, 1))

    z1 = jnp.zeros(1, jnp.int32)
    z2 = jnp.zeros(2, jnp.int32)
    t1 = jnp.full(1, 8, jnp.int32)
    t2 = jnp.full(2, 8, jnp.int32)
    idx_buf = jnp.concatenate([
        bond_indices[:, 0], z1, bond_indices[:, 1], z1, bond_type, t1,
        angle_indices[:, 0], z2, angle_indices[:, 1], z2,
        angle_indices[:, 2], z2, angle_type, t2,
        improper_indices[:, 0], improper_indices[:, 1],
        improper_indices[:, 2], improper_indices[:, 3], dih_type,
        sbbs_indices[:, 0], z1, sbbs_indices[:, 1], z1,
        sbbs_indices[:, 2], z1, sbbs_indices[:, 3], z1,
        bbbb_indices[:, 0], z1, bbbb_indices[:, 1], z1,
        bbbb_indices[:, 2], z1, bbbb_indices[:, 3], z1,
    ]).astype(jnp.int32)
    f1 = jnp.zeros(1, jnp.float32)
    f8 = jnp.zeros(8, jnp.float32)
    f12 = jnp.zeros(12, jnp.float32)
    flt_buf = jnp.concatenate([
        bond_k, f8, bond_equ, f8, angle_k, f8, angle_equ, f8,
        dih_k, f12, dih_equ, f12,
        sbbs_const, f1, sbbs_phase, f1, sbbs_mul, f1,
        bbbb_const, f1, bbbb_phase, f1, bbbb_mul, f1,
    ]).astype(jnp.float32)

    e_nb = _pairwise(pos_t, bead_types, bead_radii, dispertion_const)
    e_sc = _sc_bonded(pos.reshape(b, 3 * n), idx_buf, flt_buf)
    return e_nb + e_sc.reshape(b, 2, 16).sum(axis=(1, 2))


# speedup vs baseline: 1.1031x; 1.1031x over previous
import functools

import jax
import jax.numpy as jnp
from jax import lax
from jax.experimental import pallas as pl
from jax.experimental.pallas import tpu as pltpu
from jax.experimental.pallas import tpu_sc as plsc

N_ATOMS = 1024
R_MAX2 = 100.0


_PB = N_ATOMS
_PI = 256
_PS = 512
_PBB = 256
_OFF_B0 = 0
_OFF_B1 = _OFF_B0 + _PB
_OFF_BT = _OFF_B1 + _PB
_OFF_A0 = _OFF_BT + _PB
_OFF_A1 = _OFF_A0 + _PB
_OFF_A2 = _OFF_A1 + _PB
_OFF_AT = _OFF_A2 + _PB
_OFF_I0 = _OFF_AT + _PB
_OFF_I1 = _OFF_I0 + _PI
_OFF_I2 = _OFF_I1 + _PI
_OFF_I3 = _OFF_I2 + _PI
_OFF_IT = _OFF_I3 + _PI
_OFF_S0 = _OFF_IT + _PI
_OFF_S1 = _OFF_S0 + _PS
_OFF_S2 = _OFF_S1 + _PS
_OFF_S3 = _OFF_S2 + _PS
_OFF_G0 = _OFF_S3 + _PS
_OFF_G1 = _OFF_G0 + _PBB
_OFF_G2 = _OFF_G1 + _PBB
_OFF_G3 = _OFF_G2 + _PBB
_IDX_LEN = _OFF_G3 + _PBB

_OFF_TBK = 0
_OFF_TBE = 16
_OFF_TAK = 32
_OFF_TAE = 48
_OFF_TDK = 64
_OFF_TDE = 80
_OFF_SC = 96
_OFF_SP = _OFF_SC + _PS
_OFF_SM = _OFF_SP + _PS
_OFF_GC = _OFF_SM + _PS
_OFF_GP = _OFF_GC + _PBB
_OFF_GM = _OFF_GP + _PBB
_FLT_LEN = _OFF_GM + _PBB

_PI_F = 3.14159265358979323846


def _rsqrt16(s):
    s = jnp.maximum(s, 1e-30)
    i = lax.bitcast_convert_type(s, jnp.int32)
    i = 0x5F3759DF - lax.shift_right_logical(i, 1)
    y = lax.bitcast_convert_type(i, jnp.float32)
    for _ in range(3):
        y = y * (1.5 - 0.5 * s * y * y)
    return y


def _sqrt16(s):
    return s * _rsqrt16(s)


def _atan16(a):
    s = a * a
    p = jnp.float32(0.0028662257)
    p = p * s + jnp.float32(-0.0161657367)
    p = p * s + jnp.float32(0.0429096138)
    p = p * s + jnp.float32(-0.0752896400)
    p = p * s + jnp.float32(0.1065626393)
    p = p * s + jnp.float32(-0.1420889944)
    p = p * s + jnp.float32(0.1999355085)
    p = p * s + jnp.float32(-0.3333314528)
    p = p * s + jnp.float32(1.0)
    return a * p


def _atan2_16(y, x):
    ax = jnp.abs(x)
    ay = jnp.abs(y)
    mx = jnp.maximum(ax, ay)
    mn = jnp.minimum(ax, ay)
    a = mn / jnp.maximum(mx, 1e-30)
    r = _atan16(a)
    r = jnp.where(ay > ax, _PI_F / 2.0 - r, r)
    r = jnp.where(x < 0.0, _PI_F - r, r)
    return jnp.where(y < 0.0, -r, r)


def _cos16(t):
    q = t * jnp.float32(1.0 / (2.0 * _PI_F))
    q = q + jnp.where(q >= 0.0, 0.5, -0.5)
    k = q.astype(jnp.int32).astype(jnp.float32)
    t = t - k * jnp.float32(2.0 * _PI_F)
    s = t * t
    p = jnp.float32(1.0 / 479001600.0)
    p = p * s + jnp.float32(-1.0 / 3628800.0)
    p = p * s + jnp.float32(1.0 / 40320.0)
    p = p * s + jnp.float32(-1.0 / 720.0)
    p = p * s + jnp.float32(1.0 / 24.0)
    p = p * s + jnp.float32(-0.5)
    p = p * s + jnp.float32(1.0)
    return p


def _gather3(pos_v, idx):
    base = idx * 3
    return (plsc.load_gather(pos_v, [base]),
            plsc.load_gather(pos_v, [base + 1]),
            plsc.load_gather(pos_v, [base + 2]))


def _dih16(p0, p1, p2, p3):
    b0 = tuple(p0[c] - p1[c] for c in range(3))
    b1 = tuple(p2[c] - p1[c] for c in range(3))
    b2 = tuple(p3[c] - p2[c] for c in range(3))
    s1 = b1[0] * b1[0] + b1[1] * b1[1] + b1[2] * b1[2]
    n1 = _sqrt16(s1)
    m = 1.0 / (n1 + 1e-9)
    b1n = tuple(b1[c] * m for c in range(3))
    d0 = b0[0] * b1n[0] + b0[1] * b1n[1] + b0[2] * b1n[2]
    d2 = b2[0] * b1n[0] + b2[1] * b1n[1] + b2[2] * b1n[2]
    v = tuple(b0[c] - d0 * b1n[c] for c in range(3))
    w = tuple(b2[c] - d2 * b1n[c] for c in range(3))
    xx = v[0] * w[0] + v[1] * w[1] + v[2] * w[2]
    c0 = b1n[1] * v[2] - b1n[2] * v[1]
    c1 = b1n[2] * v[0] - b1n[0] * v[2]
    c2 = b1n[0] * v[1] - b1n[1] * v[0]
    yy = c0 * w[0] + c1 * w[1] + c2 * w[2]
    return _atan2_16(yy, xx)


def _sc_body(pos_hbm, idx_hbm, flt_hbm, out_hbm, pos_v, idx_v, flt_v, acc_v):
    nc = 2
    wid = lax.axis_index("s") * nc + lax.axis_index("c")
    frame = wid // 2
    half = wid % 2
    pltpu.sync_copy(pos_hbm.at[frame], pos_v)
    pltpu.sync_copy(idx_hbm, idx_v)
    pltpu.sync_copy(flt_hbm, flt_v)
    iota = lax.iota(jnp.int32, 16)

    def bonds(i, acc):
        t = half * (_PB // 2) + i * 16 + iota
        ia = plsc.load_gather(idx_v, [t + _OFF_B0])
        ib = plsc.load_gather(idx_v, [t + _OFF_B1])
        ty = plsc.load_gather(idx_v, [t + _OFF_BT])
        k = plsc.load_gather(flt_v, [ty + _OFF_TBK])
        e = plsc.load_gather(flt_v, [ty + _OFF_TBE])
        pa = _gather3(pos_v, ia)
        pb = _gather3(pos_v, ib)
        dx, dy, dz = (pa[c] - pb[c] for c in range(3))
        d = _sqrt16(dx * dx + dy * dy + dz * dz)
        return acc + 0.5 * k * (d - e) * (d - e)

    def angles(i, acc):
        t = half * (_PB // 2) + i * 16 + iota
        i0 = plsc.load_gather(idx_v, [t + _OFF_A0])
        i1 = plsc.load_gather(idx_v, [t + _OFF_A1])
        i2 = plsc.load_gather(idx_v, [t + _OFF_A2])
        ty = plsc.load_gather(idx_v, [t + _OFF_AT])
        k = plsc.load_gather(flt_v, [ty + _OFF_TAK])
        e = plsc.load_gather(flt_v, [ty + _OFF_TAE])
        p0 = _gather3(pos_v, i0)
        p1 = _gather3(pos_v, i1)
        p2 = _gather3(pos_v, i2)
        v1 = tuple(p0[c] - p1[c] for c in range(3))
        v2 = tuple(p2[c] - p1[c] for c in range(3))
        s1 = v1[0] * v1[0] + v1[1] * v1[1] + v1[2] * v1[2]
        s2 = v2[0] * v2[0] + v2[1] * v2[1] + v2[2] * v2[2]
        n1 = _sqrt16(s1) + 1e-9
        n2 = _sqrt16(s2) + 1e-9
        dot = v1[0] * v2[0] + v1[1] * v2[1] + v1[2] * v2[2]
        c = jnp.clip(dot / (n1 * n2), -1.0 + 1e-7, 1.0 - 1e-7)
        ang = _atan2_16(_sqrt16((1.0 - c) * (1.0 + c)), c)
        return acc + 0.5 * k * (ang - e) * (ang - e)

    def improper(i, acc):
        t = half * (_PI // 2) + i * 16 + iota
        p0 = _gather3(pos_v, plsc.load_gather(idx_v, [t + _OFF_I0]))
        p1 = _gather3(pos_v, plsc.load_gather(idx_v, [t + _OFF_I1]))
        p2 = _gather3(pos_v, plsc.load_gather(idx_v, [t + _OFF_I2]))
        p3 = _gather3(pos_v, plsc.load_gather(idx_v, [t + _OFF_I3]))
        ty = plsc.load_gather(idx_v, [t + _OFF_IT])
        k = plsc.load_gather(flt_v, [ty + _OFF_TDK])
        e = plsc.load_gather(flt_v, [ty + _OFF_TDE])
        phi = _dih16(p0, p1, p2, p3)
        return acc + 0.5 * k * (phi - e) * (phi - e)

    def sbbs(i, acc):
        t = half * (_PS // 2) + i * 16 + iota
        p0 = _gather3(pos_v, plsc.load_gather(idx_v, [t + _OFF_S0]))
        p1 = _gather3(pos_v, plsc.load_gather(idx_v, [t + _OFF_S1]))
        p2 = _gather3(pos_v, plsc.load_gather(idx_v, [t + _OFF_S2]))
        p3 = _gather3(pos_v, plsc.load_gather(idx_v, [t + _OFF_S3]))
        const = plsc.load_gather(flt_v, [t + _OFF_SC])
        phase = plsc.load_gather(flt_v, [t + _OFF_SP])
        mul = plsc.load_gather(flt_v, [t + _OFF_SM])
        phi = _dih16(p0, p1, p2, p3)
        return acc + const * (1.0 + _cos16(mul * phi - phase))

    def bbbb(i, acc):
        t = half * (_PBB // 2) + i * 16 + iota
        p0 = _gather3(pos_v, plsc.load_gather(idx_v, [t + _OFF_G0]))
        p1 = _gather3(pos_v, plsc.load_gather(idx_v, [t + _OFF_G1]))
        p2 = _gather3(pos_v, plsc.load_gather(idx_v, [t + _OFF_G2]))
        p3 = _gather3(pos_v, plsc.load_gather(idx_v, [t + _OFF_G3]))
        const = plsc.load_gather(flt_v, [t + _OFF_GC])
        phase = plsc.load_gather(flt_v, [t + _OFF_GP])
        mul = plsc.load_gather(flt_v, [t + _OFF_GM])
        phi = _dih16(p0, p1, p2, p3)
        return acc + const * (1.0 + _cos16(mul * phi - phase))

    acc = jnp.zeros((16,), jnp.float32)
    acc = lax.fori_loop(0, _PB // 32, bonds, acc)
    acc = lax.fori_loop(0, _PB // 32, angles, acc)
    acc = lax.fori_loop(0, _PI // 32, improper, acc)
    acc = lax.fori_loop(0, _PS // 32, sbbs, acc)
    acc = lax.fori_loop(0, _PBB // 32, bbbb, acc)
    acc_v[...] = acc
    pltpu.sync_copy(acc_v, out_hbm.at[wid])


def _sc_bonded(pos_flat, idx_buf, flt_buf):
    mesh = plsc.VectorSubcoreMesh(core_axis_name="c", subcore_axis_name="s")
    return pl.kernel(
        _sc_body,
        out_type=jax.ShapeDtypeStruct((32, 16), jnp.float32),
        mesh=mesh,
        scratch_types=[
            pltpu.VMEM((3 * N_ATOMS,), jnp.float32),
            pltpu.VMEM((_IDX_LEN,), jnp.int32),
            pltpu.VMEM((_FLT_LEN,), jnp.float32),
            pltpu.VMEM((16,), jnp.float32),
        ],
        compiler_params=pltpu.CompilerParams(needs_layout_passes=False),
    )(pos_flat, idx_buf, flt_buf)


def _onehot_lookup(type_plane, table_ref, n):
    out = jnp.zeros_like(type_plane, dtype=jnp.float32)
    for k in range(n):
        out = jnp.where(type_plane == k, table_ref[0, k], out)
    return out


def _pw_kernel(pos_ref, bt_ref, br_ref, disp_ref, out_ref):
    x = pos_ref[0]
    n = x.shape[1]
    bead_types = bt_ref[0:1, :]
    radii_row = _onehot_lookup(bead_types, br_ref, br_ref.shape[1])
    sq_row = jnp.sum(x * x, axis=0, keepdims=True)
    ones = jnp.ones((1, n), jnp.float32)
    lhs_r2 = jnp.concatenate([x, sq_row, ones], axis=0)
    rhs_r2 = jnp.concatenate([-2.0 * x, ones, sq_row], axis=0)
    lhs_sg = jnp.concatenate([radii_row, ones], axis=0)
    rhs_sg = jnp.concatenate([ones, radii_row], axis=0)
    tile = 128
    row = lax.broadcasted_iota(jnp.int32, (tile, tile), 0)
    col = lax.broadcasted_iota(jnp.int32, (tile, tile), 1)
    tri = col > row + 2
    tri2 = col + tile > row + 2
    e_nb = jnp.float32(0.0)
    for ti in range(n // tile):
        lo = ti * tile

        def rep_block(cl, cw, extra_mask=None):
            r2 = lax.dot_general(lhs_r2[:, lo:lo + tile],
                                 rhs_r2[:, cl:cl + cw],
                                 (((0,), (0,)), ((), ())),
                                 preferred_element_type=jnp.float32)
            sig = lax.dot_general(lhs_sg[:, lo:lo + tile],
                                  rhs_sg[:, cl:cl + cw],
                                  (((0,), (0,)), ((), ())),
                                  preferred_element_type=jnp.float32)
            r2 = jnp.maximum(r2, 1e-6).astype(jnp.bfloat16)
            sig = sig.astype(jnp.bfloat16)
            sig2 = sig * sig
            t3 = sig2 / (r2 + sig2)
            rep = t3 * t3 * t3
            pmask = r2 < jnp.bfloat16(R_MAX2)
            if extra_mask is not None:
                pmask = pmask & extra_mask
            return jnp.sum(jnp.where(pmask, rep, jnp.bfloat16(0.0)),
                           dtype=jnp.float32)

        e_nb = e_nb + rep_block(lo, tile, tri)
        if lo + tile < n:
            e_nb = e_nb + rep_block(lo + tile, tile, tri2)
        if lo + 2 * tile < n:
            e_nb = e_nb + rep_block(lo + 2 * tile, n - lo - 2 * tile)
    out_ref[...] = jnp.full((1, 1, 1), disp_ref[0, 0] * e_nb, jnp.float32)


def _pairwise(pos_t, bead_types, bead_radii, disp):
    b, _, n = pos_t.shape
    out = pl.pallas_call(
        _pw_kernel,
        grid=(b,),
        in_specs=[
            pl.BlockSpec((1, 3, n), lambda i: (i, 0, 0)),
            pl.BlockSpec((1, n), lambda i: (0, 0)),
            pl.BlockSpec((1, 16), lambda i: (0, 0)),
            pl.BlockSpec((1, 1), lambda i: (0, 0)),
        ],
        out_specs=pl.BlockSpec((1, 1, 1), lambda i: (i, 0, 0)),
        out_shape=jax.ShapeDtypeStruct((b, 1, 1), jnp.float32),
    )(pos_t, bead_types.reshape(1, n), bead_radii.reshape(1, -1),
      disp.reshape(1, 1))
    return out.reshape(b)


def _pad_to(a, n, value=0):
    return jnp.pad(a, (0, n - a.shape[0]), constant_values=value)


@jax.jit
def kernel(pos, bond_k, angle_k, dih_k, sbbs_phase, sbbs_const, bbbb_phase,
           bbbb_const, bead_radii, dispertion_const, bond_equ, angle_equ,
           dih_equ, bond_indices, bond_type, angle_indices, angle_type,
           improper_indices, dih_type, sbbs_indices, sbbs_mul, bbbb_indices,
           bbbb_mul, bead_types):
    b, n, _ = pos.shape
    pos_t = jnp.transpose(pos, (0, 2, 1))

    z1 = jnp.zeros(1, jnp.int32)
    z2 = jnp.zeros(2, jnp.int32)
    t1 = jnp.full(1, 8, jnp.int32)
    t2 = jnp.full(2, 8, jnp.int32)
    idx_buf = jnp.concatenate([
        bond_indices[:, 0], z1, bond_indices[:, 1], z1, bond_type, t1,
        angle_indices[:, 0], z2, angle_indices[:, 1], z2,
        angle_indices[:, 2], z2, angle_type, t2,
        improper_indices[:, 0], improper_indices[:, 1],
        improper_indices[:, 2], improper_indices[:, 3], dih_type,
        sbbs_indices[:, 0], z1, sbbs_indices[:, 1], z1,
        sbbs_indices[:, 2], z1, sbbs_indices[:, 3], z1,
        bbbb_indices[:, 0], z1, bbbb_indices[:, 1], z1,
        bbbb_indices[:, 2], z1, bbbb_indices[:, 3], z1,
    ]).astype(jnp.int32)
    f1 = jnp.zeros(1, jnp.float32)
    f8 = jnp.zeros(8, jnp.float32)
    f12 = jnp.zeros(12, jnp.float32)
    flt_buf = jnp.concatenate([
        bond_k, f8, bond_equ, f8, angle_k, f8, angle_equ, f8,
        dih_k, f12, dih_equ, f12,
        sbbs_const, f1, sbbs_phase, f1, sbbs_mul, f1,
        bbbb_const, f1, bbbb_phase, f1, bbbb_mul, f1,
    ]).astype(jnp.float32)

    e_nb = _pairwise(pos_t, bead_types, bead_radii, dispertion_const)
    e_sc = _sc_bonded(pos.reshape(b, 3 * n), idx_buf, flt_buf)
    return e_nb + e_sc.reshape(b, 2, 16).sum(axis=(1, 2))
